# Initial kernel scaffold; baseline (speedup 1.0000x reference)
import functools

import jax
import jax.numpy as jnp
from jax import lax
from jax.experimental import pallas as pl
from jax.experimental.pallas import tpu as pltpu
from jax.experimental.pallas import tpu_sc as plsc

NC = 2
NS = 16
NW = NC * NS
LANES = 16
W = 128


def _make_kernel(n, d):
    assert n % (NW * W) == 0
    n_win = n // (NW * W)
    mesh = plsc.VectorSubcoreMesh(core_axis_name="c", subcore_axis_name="s")

    @functools.partial(
        pl.kernel,
        mesh=mesh,
        out_type=jax.ShapeDtypeStruct((n, d), jnp.float32),
        scratch_types=[
            pltpu.VMEM((W,), jnp.int32),
            pltpu.VMEM((W, d), jnp.float32),
            pltpu.VMEM((W, d), jnp.float32),
            pltpu.SemaphoreType.DMA,
            pltpu.SemaphoreType.DMA,
        ],
    )
    def k(idx_hbm, t1_hbm, t2_hbm, out_hbm, idx_v, rows1_v, rows2_v, sem1, sem2):
        wid = lax.axis_index("s") * NC + lax.axis_index("c")
        base = wid * (n_win * W)

        @pl.loop(0, n_win)
        def _(w):
            off = base + w * W
            pltpu.sync_copy(idx_hbm.at[pl.ds(off, W)], idx_v)
            cp1 = pltpu.async_copy(t1_hbm.at[idx_v], rows1_v, sem1)
            cp2 = pltpu.async_copy(t2_hbm.at[idx_v], rows2_v, sem2)
            cp1.wait()
            cp2.wait()

            @pl.loop(0, W)
            def _(r):
                for c in range(d // LANES):
                    sl = pl.ds(c * LANES, LANES)
                    rows1_v[r, sl] = rows1_v[r, sl] + rows2_v[r, sl]

            pltpu.sync_copy(rows1_v, out_hbm.at[pl.ds(off, W)])

    return k


def kernel(x, table_1, table_2):
    b, h = x.shape
    v, d = table_1.shape
    xf = x.reshape(-1).astype(jnp.int32)
    out = _make_kernel(b * h, d)(xf, table_1, table_2)
    return out.reshape(b, h, d)


# speedup vs baseline: 1.9836x; 1.9836x over previous
import functools

import jax
import jax.numpy as jnp
from jax import lax
from jax.experimental import pallas as pl
from jax.experimental.pallas import tpu as pltpu
from jax.experimental.pallas import tpu_sc as plsc

NC = 2
NS = 16
NW = NC * NS
LANES = 16
W = 128


def _make_kernel(n, d):
    assert n % (NW * W) == 0
    n_win = n // (NW * W)
    mesh = plsc.VectorSubcoreMesh(core_axis_name="c", subcore_axis_name="s")

    @functools.partial(
        pl.kernel,
        mesh=mesh,
        out_type=jax.ShapeDtypeStruct((n, d), jnp.float32),
        compiler_params=pltpu.CompilerParams(use_tc_tiling_on_sc=False),
        scratch_types=[
            pltpu.VMEM((W,), jnp.int32),
            pltpu.VMEM((W, d), jnp.float32),
            pltpu.VMEM((W, d), jnp.float32),
            pltpu.SemaphoreType.DMA,
            pltpu.SemaphoreType.DMA,
        ],
    )
    def k(idx_hbm, t1_hbm, t2_hbm, out_hbm, idx_v, rows1_v, rows2_v, sem1, sem2):
        wid = lax.axis_index("s") * NC + lax.axis_index("c")
        base = wid * (n_win * W)

        @pl.loop(0, n_win)
        def _(w):
            off = base + w * W
            pltpu.sync_copy(idx_hbm.at[pl.ds(off, W)], idx_v)
            cp1 = pltpu.async_copy(t1_hbm.at[idx_v], rows1_v, sem1)
            cp2 = pltpu.async_copy(t2_hbm.at[idx_v], rows2_v, sem2)
            cp1.wait()
            cp2.wait()

            @pl.loop(0, W)
            def _(r):
                for c in range(d // LANES):
                    sl = pl.ds(c * LANES, LANES)
                    rows1_v[r, sl] = rows1_v[r, sl] + rows2_v[r, sl]

            pltpu.sync_copy(rows1_v, out_hbm.at[pl.ds(off, W)])

    return k


def kernel(x, table_1, table_2):
    b, h = x.shape
    v, d = table_1.shape
    xf = x.reshape(-1).astype(jnp.int32)
    out = _make_kernel(b * h, d)(xf, table_1, table_2)
    return out.reshape(b, h, d)


# speedup vs baseline: 2.8119x; 1.4176x over previous
import functools

import jax
import jax.numpy as jnp
from jax import lax
from jax.experimental import pallas as pl
from jax.experimental.pallas import tpu as pltpu
from jax.experimental.pallas import tpu_sc as plsc

NC = 2
NS = 16
NW = NC * NS
CHUNK = 128
BUF = 512
NCH = BUF // CHUNK


def _make_kernel(n, d):
    assert n % (NW * BUF) == 0
    n_blk = n // (NW * BUF)
    mesh = plsc.VectorSubcoreMesh(core_axis_name="c", subcore_axis_name="s")

    @functools.partial(
        pl.kernel,
        mesh=mesh,
        out_type=jax.ShapeDtypeStruct((n, d), jnp.float32),
        compiler_params=pltpu.CompilerParams(use_tc_tiling_on_sc=False),
        scratch_types=[
            pltpu.VMEM((BUF,), jnp.int32),
            pltpu.VMEM((BUF,), jnp.int32),
            pltpu.VMEM((BUF, d), jnp.float32),
            pltpu.VMEM((BUF, d), jnp.float32),
            pltpu.SemaphoreType.DMA,
            pltpu.SemaphoreType.DMA,
        ],
    )
    def k(idx_hbm, tab_hbm, out_hbm, idx0, idx1, rows0, rows1, sem0, sem1):
        wid = lax.axis_index("s") * NC + lax.axis_index("c")
        base = wid * (n_blk * BUF)

        def fire(blk, idx_v, rows_v, sem):
            off = base + blk * BUF
            pltpu.sync_copy(idx_hbm.at[pl.ds(off, BUF)], idx_v)
            return [
                pltpu.async_copy(
                    tab_hbm.at[idx_v.at[pl.ds(c * CHUNK, CHUNK)]],
                    rows_v.at[pl.ds(c * CHUNK, CHUNK)],
                    sem,
                )
                for c in range(NCH)
            ]

        def drain_store(blk, rows_v, handles):
            for h in handles:
                h.wait()
            off = base + blk * BUF
            pltpu.sync_copy(rows_v, out_hbm.at[pl.ds(off, BUF)])

        fire(0, idx0, rows0, sem0)

        @pl.loop(0, n_blk, step=2)
        def _(blk):
            h1 = fire(blk + 1, idx1, rows1, sem1)
            h0 = [
                pltpu.make_async_copy(
                    tab_hbm.at[idx0.at[pl.ds(c * CHUNK, CHUNK)]],
                    rows0.at[pl.ds(c * CHUNK, CHUNK)],
                    sem0,
                )
                for c in range(NCH)
            ]
            drain_store(blk, rows0, h0)

            @pl.when(blk + 2 < n_blk)
            def _():
                fire(blk + 2, idx0, rows0, sem0)

            drain_store(blk + 1, rows1, h1)

    return k


def kernel(x, table_1, table_2):
    b, h = x.shape
    v, d = table_1.shape
    summed = table_1 + table_2
    xf = x.reshape(-1).astype(jnp.int32)
    out = _make_kernel(b * h, d)(xf, summed)
    return out.reshape(b, h, d)


# speedup vs baseline: 2.9214x; 1.0390x over previous
import functools

import jax
import jax.numpy as jnp
from jax import lax
from jax.experimental import pallas as pl
from jax.experimental.pallas import tpu as pltpu
from jax.experimental.pallas import tpu_sc as plsc

NC = 2
NS = 16
NW = NC * NS
CHUNK = 128
BUF = 512
NCH = BUF // CHUNK


def _make_kernel(n, d, tw):
    assert n % (NW * BUF) == 0
    n_blk = n // (NW * BUF)
    mesh = plsc.VectorSubcoreMesh(core_axis_name="c", subcore_axis_name="s")

    @functools.partial(
        pl.kernel,
        mesh=mesh,
        out_type=jax.ShapeDtypeStruct((n, d), jnp.float32),
        compiler_params=pltpu.CompilerParams(use_tc_tiling_on_sc=False),
        scratch_types=[
            pltpu.VMEM((BUF,), jnp.int32),
            pltpu.VMEM((BUF,), jnp.int32),
            pltpu.VMEM((BUF, tw), jnp.float32),
            pltpu.VMEM((BUF, tw), jnp.float32),
            pltpu.SemaphoreType.DMA,
            pltpu.SemaphoreType.DMA,
        ],
    )
    def k(idx_hbm, tab_hbm, out_hbm, idx0, idx1, rows0, rows1, sem0, sem1):
        wid = lax.axis_index("s") * NC + lax.axis_index("c")
        base = wid * (n_blk * BUF)

        def fire(blk, idx_v, rows_v, sem):
            off = base + blk * BUF
            pltpu.sync_copy(idx_hbm.at[pl.ds(off, BUF)], idx_v)
            return [
                pltpu.async_copy(
                    tab_hbm.at[idx_v.at[pl.ds(c * CHUNK, CHUNK)]],
                    rows_v.at[pl.ds(c * CHUNK, CHUNK)],
                    sem,
                )
                for c in range(NCH)
            ]

        def drain_store(blk, rows_v, handles):
            for h in handles:
                h.wait()
            off = base + blk * BUF
            pltpu.sync_copy(rows_v.at[:, pl.ds(0, d)], out_hbm.at[pl.ds(off, BUF)])

        fire(0, idx0, rows0, sem0)

        @pl.loop(0, n_blk, step=2)
        def _(blk):
            h1 = fire(blk + 1, idx1, rows1, sem1)
            h0 = [
                pltpu.make_async_copy(
                    tab_hbm.at[idx0.at[pl.ds(c * CHUNK, CHUNK)]],
                    rows0.at[pl.ds(c * CHUNK, CHUNK)],
                    sem0,
                )
                for c in range(NCH)
            ]
            drain_store(blk, rows0, h0)

            @pl.when(blk + 2 < n_blk)
            def _():
                fire(blk + 2, idx0, rows0, sem0)

            drain_store(blk + 1, rows1, h1)

    return k


def kernel(x, table_1, table_2):
    b, h = x.shape
    v, d = table_1.shape
    summed = table_1 + table_2
    xf = jnp.transpose(x).reshape(-1).astype(jnp.int32)
    out = _make_kernel(b * h, d, summed.shape[1])(xf, summed)
    return jnp.transpose(out.reshape(h, b, d), (1, 0, 2))


# speedup vs baseline: 2.9475x; 1.0089x over previous
import functools

import jax
import jax.numpy as jnp
from jax import lax
from jax.experimental import pallas as pl
from jax.experimental.pallas import tpu as pltpu
from jax.experimental.pallas import tpu_sc as plsc

NC = 2
NS = 16
NW = NC * NS
CHUNK = 128
BUF = 640
NCH = BUF // CHUNK


def _make_kernel(n, d, tw):
    assert n % (NW * BUF) == 0
    n_blk = n // (NW * BUF)
    mesh = plsc.VectorSubcoreMesh(core_axis_name="c", subcore_axis_name="s")

    @functools.partial(
        pl.kernel,
        mesh=mesh,
        out_type=jax.ShapeDtypeStruct((n, d), jnp.float32),
        compiler_params=pltpu.CompilerParams(use_tc_tiling_on_sc=False),
        scratch_types=[
            pltpu.VMEM((BUF,), jnp.int32),
            pltpu.VMEM((BUF,), jnp.int32),
            pltpu.VMEM((BUF, tw), jnp.float32),
            pltpu.VMEM((BUF, tw), jnp.float32),
            pltpu.SemaphoreType.DMA,
            pltpu.SemaphoreType.DMA,
        ],
    )
    def k(idx_hbm, tab_hbm, out_hbm, idx0, idx1, rows0, rows1, sem0, sem1):
        wid = lax.axis_index("s") * NC + lax.axis_index("c")
        base = wid * (n_blk * BUF)

        def fire(blk, idx_v, rows_v, sem):
            off = base + blk * BUF
            pltpu.sync_copy(idx_hbm.at[pl.ds(off, BUF)], idx_v)
            return [
                pltpu.async_copy(
                    tab_hbm.at[idx_v.at[pl.ds(c * CHUNK, CHUNK)]],
                    rows_v.at[pl.ds(c * CHUNK, CHUNK)],
                    sem,
                )
                for c in range(NCH)
            ]

        def drain_store(blk, rows_v, handles):
            for h in handles:
                h.wait()
            off = base + blk * BUF
            pltpu.sync_copy(rows_v.at[:, pl.ds(0, d)], out_hbm.at[pl.ds(off, BUF)])

        fire(0, idx0, rows0, sem0)

        @pl.loop(0, n_blk, step=2)
        def _(blk):
            h1 = fire(blk + 1, idx1, rows1, sem1)
            h0 = [
                pltpu.make_async_copy(
                    tab_hbm.at[idx0.at[pl.ds(c * CHUNK, CHUNK)]],
                    rows0.at[pl.ds(c * CHUNK, CHUNK)],
                    sem0,
                )
                for c in range(NCH)
            ]
            drain_store(blk, rows0, h0)

            @pl.when(blk + 2 < n_blk)
            def _():
                fire(blk + 2, idx0, rows0, sem0)

            drain_store(blk + 1, rows1, h1)

    return k


def kernel(x, table_1, table_2):
    b, h = x.shape
    v, d = table_1.shape
    summed = table_1 + table_2
    xf = jnp.transpose(x).reshape(-1).astype(jnp.int32)
    out = _make_kernel(b * h, d, summed.shape[1])(xf, summed)
    return jnp.transpose(out.reshape(h, b, d), (1, 0, 2))
